# baseline (speedup 1.0000x reference)
import numpy as np
import jax
import jax.numpy as jnp
from jax import lax
from jax.experimental import pallas as pl
from jax.experimental.pallas import tpu as pltpu
from jax.experimental.pallas import tpu_sc as plsc

EMBED = 64
BATCH = 4096
NSAMP = 5
RANGE_WIDTH = 50000
TOTAL_DRAWS = 2 * BATCH * NSAMP

NC = 2
NSUB = 16
NW = NC * NSUB
RPW = BATCH // NW
NPW = RPW * NSAMP

_BD = np.random.RandomState(0).randint(
    0, RANGE_WIDTH, size=(TOTAL_DRAWS,)).astype(np.int32)
_BD2R = np.concatenate([_BD, _BD + RANGE_WIDTH]).astype(np.int32).reshape(-1, 16)

_RBLK = 32
_CBLK = BATCH // _RBLK


def _prep_body(t_ref, dn_ref, rn_ref, dc_ref, rc_ref, sub0_ref):
    t = t_ref[...]
    m0 = (t == 0)
    m0f = m0.astype(jnp.float32)
    ri = lax.broadcasted_iota(jnp.int32, (_CBLK, _CBLK), 0)
    ci = lax.broadcasted_iota(jnp.int32, (_CBLK, _CBLK), 1)
    upper = (ri <= ci).astype(jnp.float32)
    crow = jnp.dot(m0f, upper, preferred_element_type=jnp.float32,
                   precision=lax.Precision.HIGHEST)
    s = crow[:, _CBLK - 1:_CBLK]
    ri2 = lax.broadcasted_iota(jnp.int32, (_RBLK, _RBLK), 0)
    ci2 = lax.broadcasted_iota(jnp.int32, (_RBLK, _RBLK), 1)
    lower = (ri2 > ci2).astype(jnp.float32)
    off = jnp.dot(lower, jnp.broadcast_to(s, (_RBLK, _CBLK)),
                  preferred_element_type=jnp.float32,
                  precision=lax.Precision.HIGHEST)[:, 0:1]
    cum = (off + crow).astype(jnp.int32)
    g0 = lax.broadcasted_iota(jnp.int32, (_RBLK, _CBLK), 0)
    g1 = lax.broadcasted_iota(jnp.int32, (_RBLK, _CBLK), 1)
    gidx = g0 * _CBLK + g1
    rank = jnp.where(m0, cum - 1, gidx - cum)
    sub0 = cum[_RBLK - 1:_RBLK, _CBLK - 1:_CBLK]
    n0s = sub0 * NSAMP
    n1s = (BATCH - sub0) * NSAMP
    noise_start = jnp.where(m0, 0, 2 * n0s)
    cp_start = jnp.where(m0, n0s, 2 * n0s + n1s)
    posn = (noise_start + rank * NSAMP).astype(jnp.float32)
    posc = (cp_start + rank * NSAMP).astype(jnp.float32)
    tf = t.astype(jnp.float32)

    re = lax.broadcasted_iota(jnp.int32, (_RBLK * NSAMP, _RBLK), 0)
    ce = lax.broadcasted_iota(jnp.int32, (_RBLK * NSAMP, _RBLK), 1)
    exp_mat = (re // NSAMP == ce).astype(jnp.float32)
    svec = (lax.broadcasted_iota(jnp.int32, (_RBLK * NSAMP, _CBLK), 0)
            % NSAMP).astype(jnp.float32)
    posn3 = jnp.dot(exp_mat, posn, preferred_element_type=jnp.float32,
                    precision=lax.Precision.HIGHEST) + svec
    posc3 = jnp.dot(exp_mat, posc, preferred_element_type=jnp.float32,
                    precision=lax.Precision.HIGHEST) + svec
    t3 = jnp.dot(exp_mat, tf, preferred_element_type=jnp.float32,
                    precision=lax.Precision.HIGHEST)
    pn = (posn3 + TOTAL_DRAWS * t3).astype(jnp.int32)
    pc = (posc3 + TOTAL_DRAWS * (1.0 - t3)).astype(jnp.int32)
    dn_ref[...] = pn // 16
    rn_ref[...] = pn % 16
    dc_ref[...] = pc // 16
    rc_ref[...] = pc % 16
    sub0_ref[...] = sub0


def _prep(types2d):
    i32 = jnp.int32
    return pl.pallas_call(
        _prep_body,
        out_shape=(
            jax.ShapeDtypeStruct((_RBLK * NSAMP, _CBLK), i32),
            jax.ShapeDtypeStruct((_RBLK * NSAMP, _CBLK), i32),
            jax.ShapeDtypeStruct((_RBLK * NSAMP, _CBLK), i32),
            jax.ShapeDtypeStruct((_RBLK * NSAMP, _CBLK), i32),
            jax.ShapeDtypeStruct((1, 1), i32),
        ),
    )(types2d)


def _sc_a_body(in_emb, bd2_hbm, idx_in_hbm, dn_hbm, rn_hbm,
               ga_inp, ga_noise,
               dn_v, rn_v, idxn_v, idxmi_v, bdr_n, rows_mi, rows_n,
               sem_s, sem_b, sem_r):
    c = lax.axis_index("c")
    s = lax.axis_index("s")
    wid = s * NC + c
    base = wid * RPW

    ph0 = [
        pltpu.async_copy(dn_hbm.at[pl.ds(wid * NSAMP, NSAMP)], dn_v, sem_s),
        pltpu.async_copy(rn_hbm.at[pl.ds(wid * NSAMP, NSAMP)], rn_v, sem_s),
        pltpu.async_copy(idx_in_hbm.at[pl.ds(base, RPW)], idxmi_v, sem_s),
    ]
    for cp in ph0:
        cp.wait()
    row_cps = [pltpu.async_copy(in_emb.at[idxmi_v], rows_mi, sem_r)]
    bd_cps = [pltpu.async_copy(bd2_hbm.at[dn_v.at[sl]], bdr_n.at[sl], sem_b)
              for sl in range(NSAMP)]
    for cp in bd_cps:
        cp.wait()
    lane = lax.broadcasted_iota(jnp.int32, (16,), 0)
    for sl in range(NSAMP):
        for ch in range(RPW // 16):
            sl16 = jnp.zeros((16,), jnp.int32) + sl
            idxn_v[sl, pl.ds(ch * 16, 16)] = plsc.load_gather(
                bdr_n, [sl16, lane + ch * 16, rn_v[sl, pl.ds(ch * 16, 16)]])
    for sl in range(NSAMP):
        row_cps.append(
            pltpu.async_copy(in_emb.at[idxn_v.at[sl]], rows_n.at[sl], sem_r))
    for cp in row_cps:
        cp.wait()
    wb = [pltpu.async_copy(rows_mi, ga_inp.at[wid], sem_s),
          pltpu.async_copy(rows_n, ga_noise.at[wid], sem_s)]
    for cp in wb:
        cp.wait()


def _sc_a(in_emb, bd2r, idx_in, dn, rn):
    f32 = jnp.float32
    i32 = jnp.int32
    return pl.kernel(
        _sc_a_body,
        out_type=[
            jax.ShapeDtypeStruct((NW, RPW, EMBED), f32),
            jax.ShapeDtypeStruct((NW, NSAMP, RPW, EMBED), f32),
        ],
        mesh=plsc.VectorSubcoreMesh(core_axis_name="c", subcore_axis_name="s"),
        compiler_params=pltpu.CompilerParams(needs_layout_passes=False,
                                             use_tc_tiling_on_sc=False),
        scratch_types=[
            pltpu.VMEM((NSAMP, RPW), i32),
            pltpu.VMEM((NSAMP, RPW), i32),
            pltpu.VMEM((NSAMP, RPW), i32),
            pltpu.VMEM((RPW,), i32),
            pltpu.VMEM((NSAMP, RPW, 16), i32),
            pltpu.VMEM((RPW, EMBED), f32),
            pltpu.VMEM((NSAMP, RPW, EMBED), f32),
            pltpu.SemaphoreType.DMA,
            pltpu.SemaphoreType.DMA,
            pltpu.SemaphoreType.DMA,
        ],
    )(in_emb, bd2r, idx_in, dn, rn)


def _sc_b_body(out_emb, bd2_hbm, idx_out_hbm, types_hbm, w0_hbm, w1_hbm,
               dc_hbm, rc_hbm, ga_inp, ga_noise,
               zp_hbm,
               dc_v, rc_v, idxc_v, idxmo_v, types_v, w0_v, w1_v,
               bdr_c, rows_mi, rows_mo, rows_n, rows_c, zp_v,
               sem_s, sem_b, sem_r, semn0, semn1):
    c = lax.axis_index("c")
    s = lax.axis_index("s")
    wid = s * NC + c
    base = wid * RPW

    ph0 = [
        pltpu.async_copy(dc_hbm.at[pl.ds(wid * NSAMP, NSAMP)], dc_v, sem_s),
        pltpu.async_copy(rc_hbm.at[pl.ds(wid * NSAMP, NSAMP)], rc_v, sem_s),
        pltpu.async_copy(idx_out_hbm.at[pl.ds(base, RPW)], idxmo_v, sem_s),
        pltpu.async_copy(types_hbm.at[pl.ds(base, RPW)], types_v, sem_s),
        pltpu.async_copy(w0_hbm, w0_v, sem_s),
        pltpu.async_copy(w1_hbm, w1_v, sem_s),
        pltpu.async_copy(ga_inp.at[wid], rows_mi, sem_r),
    ]
    semn = [semn0, semn1]

    def fire_n(sl):
        return pltpu.async_copy(ga_noise.at[wid, sl], rows_n.at[sl % 2],
                                semn[sl % 2])

    ncps = [fire_n(0), fire_n(1)]
    for cp in ph0:
        cp.wait()
    main_cp = pltpu.async_copy(out_emb.at[idxmo_v], rows_mo, sem_r)
    bd_cps = [pltpu.async_copy(bd2_hbm.at[dc_v.at[sl]], bdr_c.at[sl], sem_b)
              for sl in range(NSAMP)]
    for cp in bd_cps:
        cp.wait()
    lane = lax.broadcasted_iota(jnp.int32, (16,), 0)
    for sl in range(NSAMP):
        for ch in range(RPW // 16):
            sl16 = jnp.zeros((16,), jnp.int32) + sl
            idxc_v[sl, pl.ds(ch * 16, 16)] = plsc.load_gather(
                bdr_c, [sl16, lane + ch * 16, rc_v[sl, pl.ds(ch * 16, 16)]])
    cp_cps = [pltpu.async_copy(out_emb.at[idxc_v.at[sl]], rows_c.at[sl], sem_b)
              for sl in range(NSAMP)]
    main_cp.wait()

    w0c = [w0_v[pl.ds(ci * 16, 16)] for ci in range(EMBED // 16)]
    w1c = [w1_v[pl.ds(ci * 16, 16)] for ci in range(EMBED // 16)]
    zero = jnp.zeros((16,), jnp.float32)
    nchunk = EMBED // 16

    def wsel_chunks(r):
        tm = plsc.load_gather(types_v, [jnp.zeros((16,), jnp.int32) + r])
        return [jnp.where(tm != 0, w1c[ci], w0c[ci]) for ci in range(nchunk)]

    def pass_a(r, carry):
        ws = wsel_chunks(r)
        zt = zero
        ninp = zero
        noutp = zero
        for ci in range(nchunk):
            ic = rows_mi[r, pl.ds(ci * 16, 16)]
            oc = rows_mo[r, pl.ds(ci * 16, 16)]
            zt = zt + ic * (oc * ws[ci])
            ninp = ninp + ic * ic
            noutp = noutp + oc * oc
        zp_v[r, 0, :] = zt
        zp_v[r, 11, :] = ninp
        zp_v[r, 12, :] = noutp
        zp_v[r, 13, :] = zero
        return carry

    lax.fori_loop(0, RPW, pass_a, 0)

    def make_slab_n(sl, buf):
        def body(r, carry):
            ws = wsel_chunks(r)
            zu = zero
            nn = zero
            for ci in range(nchunk):
                nc = rows_n[buf, r, pl.ds(ci * 16, 16)]
                oc = rows_mo[r, pl.ds(ci * 16, 16)]
                zu = zu + nc * (oc * ws[ci])
                nn = nn + nc * nc
            zp_v[r, 1 + sl, :] = zu
            zp_v[r, 13, :] = zp_v[r, 13, :] + nn
            return carry
        return body

    for sl in range(NSAMP):
        ncps[sl].wait()
        lax.fori_loop(0, RPW, make_slab_n(sl, sl % 2), 0)
        if sl + 2 < NSAMP:
            ncps.append(fire_n(sl + 2))

    for cp in cp_cps:
        cp.wait()

    def make_slab_c(sl):
        def body(r, carry):
            ws = wsel_chunks(r)
            zv = zero
            ncp = zero
            for ci in range(nchunk):
                cc = rows_c[sl, r, pl.ds(ci * 16, 16)]
                ic = rows_mi[r, pl.ds(ci * 16, 16)]
                zv = zv + cc * (ic * ws[ci])
                ncp = ncp + cc * cc
            zp_v[r, 6 + sl, :] = zv
            if sl == 0:
                zp_v[r, 14, :] = ncp
            else:
                zp_v[r, 14, :] = zp_v[r, 14, :] + ncp
            return carry
        return body

    for sl in range(NSAMP):
        lax.fori_loop(0, RPW, make_slab_c(sl), 0)

    pltpu.sync_copy(zp_v, zp_hbm.at[wid])


def _sc_b(out_emb, bd2r, idx_out, types, w0, w1, dc, rc, ga_inp, ga_noise):
    f32 = jnp.float32
    i32 = jnp.int32
    return pl.kernel(
        _sc_b_body,
        out_type=jax.ShapeDtypeStruct((NW, RPW, 15, 16), f32),
        mesh=plsc.VectorSubcoreMesh(core_axis_name="c", subcore_axis_name="s"),
        compiler_params=pltpu.CompilerParams(needs_layout_passes=False,
                                             use_tc_tiling_on_sc=False),
        scratch_types=[
            pltpu.VMEM((NSAMP, RPW), i32),
            pltpu.VMEM((NSAMP, RPW), i32),
            pltpu.VMEM((NSAMP, RPW), i32),
            pltpu.VMEM((RPW,), i32),
            pltpu.VMEM((RPW,), i32),
            pltpu.VMEM((EMBED,), f32),
            pltpu.VMEM((EMBED,), f32),
            pltpu.VMEM((NSAMP, RPW, 16), i32),
            pltpu.VMEM((RPW, EMBED), f32),
            pltpu.VMEM((RPW, EMBED), f32),
            pltpu.VMEM((2, RPW, EMBED), f32),
            pltpu.VMEM((NSAMP, RPW, EMBED), f32),
            pltpu.VMEM((RPW, 15, 16), f32),
            pltpu.SemaphoreType.DMA,
            pltpu.SemaphoreType.DMA,
            pltpu.SemaphoreType.DMA,
            pltpu.SemaphoreType.DMA,
            pltpu.SemaphoreType.DMA,
        ],
    )(out_emb, bd2r, idx_out, types, w0, w1, dc, rc, ga_inp, ga_noise)


def _logsig(x):
    return jnp.minimum(x, 0.0) - jnp.log(1.0 + jnp.exp(-jnp.abs(x)))


def _finish_body(zp_ref, w0_ref, w1_ref, sub0_ref, out_ref):
    b = pl.program_id(0)
    blk = zp_ref[...]
    ri = lax.broadcasted_iota(jnp.int32, (15 * 16, 16), 0)
    ci = lax.broadcasted_iota(jnp.int32, (15 * 16, 16), 1)
    sel = (ri // 16 == ci).astype(jnp.float32)
    z = jnp.dot(blk, sel, preferred_element_type=jnp.float32,
                precision=lax.Precision.HIGHEST)
    k = lax.broadcasted_iota(jnp.int32, (1, 16), 1)
    wlog = jnp.where(k == 0, 2.0, jnp.where(k < 11, 1.0, 0.0))
    wreg = jnp.where((k >= 11) & (k < 15), 1.0, 0.0)
    val = -jnp.sum(_logsig(z) * wlog) + jnp.sum(z * wreg)

    @pl.when(b == 0)
    def _():
        out_ref[...] = jnp.zeros_like(out_ref)

    out_ref[...] += val

    @pl.when(b == 7)
    def _():
        w0 = w0_ref[...]
        w1 = w1_ref[...]
        sub0 = sub0_ref[0, 0].astype(jnp.float32)
        wterm = (sub0 * jnp.sum(w0 * w0)
                 + (jnp.float32(BATCH) - sub0) * jnp.sum(w1 * w1))
        out_ref[...] = (out_ref[...] + wterm) / (2.0 * BATCH)


def _finish(zp2, w0, w1, sub0):
    return pl.pallas_call(
        _finish_body,
        grid=(8,),
        in_specs=[
            pl.BlockSpec((BATCH // 8, 15 * 16), lambda b: (b, 0)),
            pl.BlockSpec((1, EMBED), lambda b: (0, 0)),
            pl.BlockSpec((1, EMBED), lambda b: (0, 0)),
            pl.BlockSpec((1, 1), lambda b: (0, 0)),
        ],
        out_specs=pl.BlockSpec((1, 1), lambda b: (0, 0)),
        out_shape=jax.ShapeDtypeStruct((1, 1), jnp.float32),
    )(zp2, w0, w1, sub0)


def kernel(input_labels, out_labels, num_sampled, in_embed_w, out_embed_w,
           edge_w0, edge_w1):
    del num_sampled
    types = input_labels[:, 0]
    in_ids = input_labels[:, 1]
    out_t = out_labels[:, 1]

    dn, rn, dc, rc, sub0 = _prep(types.reshape(_RBLK, _CBLK))

    bd2r = jnp.asarray(_BD2R)
    ga_inp, ga_noise = _sc_a(in_embed_w, bd2r, in_ids, dn, rn)
    zp = _sc_b(out_embed_w, bd2r, out_t, types, edge_w0, edge_w1,
               dc, rc, ga_inp, ga_noise)

    res = _finish(zp.reshape(BATCH, 15 * 16),
                  edge_w0.reshape(1, EMBED), edge_w1.reshape(1, EMBED), sub0)
    return res[0, 0]


# speedup vs baseline: 1.0415x; 1.0415x over previous
import numpy as np
import jax
import jax.numpy as jnp
from jax import lax
from jax.experimental import pallas as pl
from jax.experimental.pallas import tpu as pltpu
from jax.experimental.pallas import tpu_sc as plsc

EMBED = 64
BATCH = 4096
NSAMP = 5
RANGE_WIDTH = 50000
TOTAL_DRAWS = 2 * BATCH * NSAMP

NC = 2
NSUB = 16
NW = NC * NSUB
RPW = BATCH // NW
NPW = RPW * NSAMP

_BD = np.random.RandomState(0).randint(
    0, RANGE_WIDTH, size=(TOTAL_DRAWS,)).astype(np.int32)
_BD2R = np.concatenate([_BD, _BD + RANGE_WIDTH]).astype(np.int32).reshape(-1, 16)

_RBLK = 32
_CBLK = BATCH // _RBLK


def _prep_body(t_ref, dn_ref, rn_ref, dc_ref, rc_ref, sub0_ref):
    t = t_ref[...]
    m0 = (t == 0)
    m0f = m0.astype(jnp.float32)
    ri = lax.broadcasted_iota(jnp.int32, (_CBLK, _CBLK), 0)
    ci = lax.broadcasted_iota(jnp.int32, (_CBLK, _CBLK), 1)
    upper = (ri <= ci).astype(jnp.float32)
    crow = jnp.dot(m0f, upper, preferred_element_type=jnp.float32,
                   precision=lax.Precision.HIGHEST)
    s = crow[:, _CBLK - 1:_CBLK]
    ri2 = lax.broadcasted_iota(jnp.int32, (_RBLK, _RBLK), 0)
    ci2 = lax.broadcasted_iota(jnp.int32, (_RBLK, _RBLK), 1)
    lower = (ri2 > ci2).astype(jnp.float32)
    off = jnp.dot(lower, jnp.broadcast_to(s, (_RBLK, _CBLK)),
                  preferred_element_type=jnp.float32,
                  precision=lax.Precision.HIGHEST)[:, 0:1]
    cum = (off + crow).astype(jnp.int32)
    g0 = lax.broadcasted_iota(jnp.int32, (_RBLK, _CBLK), 0)
    g1 = lax.broadcasted_iota(jnp.int32, (_RBLK, _CBLK), 1)
    gidx = g0 * _CBLK + g1
    rank = jnp.where(m0, cum - 1, gidx - cum)
    sub0 = cum[_RBLK - 1:_RBLK, _CBLK - 1:_CBLK]
    n0s = sub0 * NSAMP
    n1s = (BATCH - sub0) * NSAMP
    noise_start = jnp.where(m0, 0, 2 * n0s)
    cp_start = jnp.where(m0, n0s, 2 * n0s + n1s)
    posn = (noise_start + rank * NSAMP).astype(jnp.float32)
    posc = (cp_start + rank * NSAMP).astype(jnp.float32)
    tf = t.astype(jnp.float32)

    re = lax.broadcasted_iota(jnp.int32, (_RBLK * NSAMP, _RBLK), 0)
    ce = lax.broadcasted_iota(jnp.int32, (_RBLK * NSAMP, _RBLK), 1)
    exp_mat = (re // NSAMP == ce).astype(jnp.float32)
    svec = (lax.broadcasted_iota(jnp.int32, (_RBLK * NSAMP, _CBLK), 0)
            % NSAMP).astype(jnp.float32)
    posn3 = jnp.dot(exp_mat, posn, preferred_element_type=jnp.float32,
                    precision=lax.Precision.HIGHEST) + svec
    posc3 = jnp.dot(exp_mat, posc, preferred_element_type=jnp.float32,
                    precision=lax.Precision.HIGHEST) + svec
    t3 = jnp.dot(exp_mat, tf, preferred_element_type=jnp.float32,
                    precision=lax.Precision.HIGHEST)
    pn = (posn3 + TOTAL_DRAWS * t3).astype(jnp.int32)
    pc = (posc3 + TOTAL_DRAWS * (1.0 - t3)).astype(jnp.int32)
    dn_ref[...] = pn // 16
    rn_ref[...] = pn % 16
    dc_ref[...] = pc // 16
    rc_ref[...] = pc % 16
    sub0_ref[...] = sub0


def _prep(types2d):
    i32 = jnp.int32
    return pl.pallas_call(
        _prep_body,
        out_shape=(
            jax.ShapeDtypeStruct((_RBLK * NSAMP, _CBLK), i32),
            jax.ShapeDtypeStruct((_RBLK * NSAMP, _CBLK), i32),
            jax.ShapeDtypeStruct((_RBLK * NSAMP, _CBLK), i32),
            jax.ShapeDtypeStruct((_RBLK * NSAMP, _CBLK), i32),
            jax.ShapeDtypeStruct((1, 1), i32),
        ),
    )(types2d)


def _sc_body(in_emb, out_emb, bd2_hbm, idx_in_hbm, idx_out_hbm, types_hbm,
             w0_hbm, w1_hbm, dn_hbm, rn_hbm, dc_hbm, rc_hbm,
             zp_hbm,
             dn_v, rn_v, dc_v, rc_v, idxn_v, idxc_v, idxmi_v, idxmo_v,
             types_v, w0_v, w1_v, bdr_n, bdr_c, rows_mi, rows_mo,
             rows_n, rows_c, zp_v,
             sem_s, sem_b, sem_r, semn0, semn1, semc0, semc1):
    c = lax.axis_index("c")
    s = lax.axis_index("s")
    wid = s * NC + c
    base = wid * RPW

    ph0 = [
        pltpu.async_copy(dn_hbm.at[pl.ds(wid * NSAMP, NSAMP)], dn_v, sem_s),
        pltpu.async_copy(rn_hbm.at[pl.ds(wid * NSAMP, NSAMP)], rn_v, sem_s),
        pltpu.async_copy(dc_hbm.at[pl.ds(wid * NSAMP, NSAMP)], dc_v, sem_s),
        pltpu.async_copy(rc_hbm.at[pl.ds(wid * NSAMP, NSAMP)], rc_v, sem_s),
        pltpu.async_copy(idx_in_hbm.at[pl.ds(base, RPW)], idxmi_v, sem_s),
        pltpu.async_copy(idx_out_hbm.at[pl.ds(base, RPW)], idxmo_v, sem_s),
        pltpu.async_copy(types_hbm.at[pl.ds(base, RPW)], types_v, sem_s),
        pltpu.async_copy(w0_hbm, w0_v, sem_s),
        pltpu.async_copy(w1_hbm, w1_v, sem_s),
    ]
    for cp in ph0:
        cp.wait()
    main_cps = [
        pltpu.async_copy(in_emb.at[idxmi_v], rows_mi, sem_r),
        pltpu.async_copy(out_emb.at[idxmo_v], rows_mo, sem_r),
    ]
    bd_cps = [pltpu.async_copy(bd2_hbm.at[dn_v.at[sl]], bdr_n.at[sl], sem_b)
              for sl in range(NSAMP)]
    bd_cps += [pltpu.async_copy(bd2_hbm.at[dc_v.at[sl]], bdr_c.at[sl], sem_b)
               for sl in range(NSAMP)]
    for cp in bd_cps:
        cp.wait()
    lane = lax.broadcasted_iota(jnp.int32, (16,), 0)
    for sl in range(NSAMP):
        for ch in range(RPW // 16):
            sl16 = jnp.zeros((16,), jnp.int32) + sl
            idxn_v[sl, pl.ds(ch * 16, 16)] = plsc.load_gather(
                bdr_n, [sl16, lane + ch * 16, rn_v[sl, pl.ds(ch * 16, 16)]])
            idxc_v[sl, pl.ds(ch * 16, 16)] = plsc.load_gather(
                bdr_c, [sl16, lane + ch * 16, rc_v[sl, pl.ds(ch * 16, 16)]])

    semn = [semn0, semn1]
    semc = [semc0, semc1]

    def fire_n(sl):
        return pltpu.async_copy(in_emb.at[idxn_v.at[sl]],
                                rows_n.at[sl % 2], semn[sl % 2])

    def fire_c(sl):
        return pltpu.async_copy(out_emb.at[idxc_v.at[sl]],
                                rows_c.at[sl % 2], semc[sl % 2])

    ncps = [fire_n(0), fire_n(1)]
    ccps = [fire_c(0), fire_c(1)]
    for cp in main_cps:
        cp.wait()

    w0c = [w0_v[pl.ds(ci * 16, 16)] for ci in range(EMBED // 16)]
    w1c = [w1_v[pl.ds(ci * 16, 16)] for ci in range(EMBED // 16)]
    zero = jnp.zeros((16,), jnp.float32)
    nchunk = EMBED // 16

    def wsel_chunks(r):
        tm = plsc.load_gather(types_v, [jnp.zeros((16,), jnp.int32) + r])
        return [jnp.where(tm != 0, w1c[ci], w0c[ci]) for ci in range(nchunk)]

    def pass_a(r, carry):
        ws = wsel_chunks(r)
        zt = zero
        ninp = zero
        noutp = zero
        for ci in range(nchunk):
            ic = rows_mi[r, pl.ds(ci * 16, 16)]
            oc = rows_mo[r, pl.ds(ci * 16, 16)]
            zt = zt + ic * (oc * ws[ci])
            ninp = ninp + ic * ic
            noutp = noutp + oc * oc
        zp_v[r, 0, :] = zt
        zp_v[r, 11, :] = ninp
        zp_v[r, 12, :] = noutp
        zp_v[r, 13, :] = zero
        zp_v[r, 14, :] = zero
        return carry

    lax.fori_loop(0, RPW, pass_a, 0)

    def make_slab_n(sl, buf):
        def body(r, carry):
            ws = wsel_chunks(r)
            zu = zero
            nn = zero
            for ci in range(nchunk):
                nc = rows_n[buf, r, pl.ds(ci * 16, 16)]
                oc = rows_mo[r, pl.ds(ci * 16, 16)]
                zu = zu + nc * (oc * ws[ci])
                nn = nn + nc * nc
            zp_v[r, 1 + sl, :] = zu
            zp_v[r, 13, :] = zp_v[r, 13, :] + nn
            return carry
        return body

    def make_slab_c(sl, buf):
        def body(r, carry):
            ws = wsel_chunks(r)
            zv = zero
            ncp = zero
            for ci in range(nchunk):
                cc = rows_c[buf, r, pl.ds(ci * 16, 16)]
                ic = rows_mi[r, pl.ds(ci * 16, 16)]
                zv = zv + cc * (ic * ws[ci])
                ncp = ncp + cc * cc
            zp_v[r, 6 + sl, :] = zv
            zp_v[r, 14, :] = zp_v[r, 14, :] + ncp
            return carry
        return body

    for sl in range(NSAMP):
        ncps[sl].wait()
        lax.fori_loop(0, RPW, make_slab_n(sl, sl % 2), 0)
        if sl + 2 < NSAMP:
            ncps.append(fire_n(sl + 2))
        ccps[sl].wait()
        lax.fori_loop(0, RPW, make_slab_c(sl, sl % 2), 0)
        if sl + 2 < NSAMP:
            ccps.append(fire_c(sl + 2))

    pltpu.sync_copy(zp_v, zp_hbm.at[wid])


def _sc_fused(in_emb, out_emb, bd2r, idx_in, idx_out, types, w0, w1,
              dn, rn, dc, rc):
    f32 = jnp.float32
    i32 = jnp.int32
    return pl.kernel(
        _sc_body,
        out_type=jax.ShapeDtypeStruct((NW, RPW, 15, 16), f32),
        mesh=plsc.VectorSubcoreMesh(core_axis_name="c", subcore_axis_name="s"),
        compiler_params=pltpu.CompilerParams(needs_layout_passes=False,
                                             use_tc_tiling_on_sc=False),
        scratch_types=[
            pltpu.VMEM((NSAMP, RPW), i32),
            pltpu.VMEM((NSAMP, RPW), i32),
            pltpu.VMEM((NSAMP, RPW), i32),
            pltpu.VMEM((NSAMP, RPW), i32),
            pltpu.VMEM((NSAMP, RPW), i32),
            pltpu.VMEM((NSAMP, RPW), i32),
            pltpu.VMEM((RPW,), i32),
            pltpu.VMEM((RPW,), i32),
            pltpu.VMEM((RPW,), i32),
            pltpu.VMEM((EMBED,), f32),
            pltpu.VMEM((EMBED,), f32),
            pltpu.VMEM((NSAMP, RPW, 16), i32),
            pltpu.VMEM((NSAMP, RPW, 16), i32),
            pltpu.VMEM((RPW, EMBED), f32),
            pltpu.VMEM((RPW, EMBED), f32),
            pltpu.VMEM((2, RPW, EMBED), f32),
            pltpu.VMEM((2, RPW, EMBED), f32),
            pltpu.VMEM((RPW, 15, 16), f32),
            pltpu.SemaphoreType.DMA,
            pltpu.SemaphoreType.DMA,
            pltpu.SemaphoreType.DMA,
            pltpu.SemaphoreType.DMA,
            pltpu.SemaphoreType.DMA,
            pltpu.SemaphoreType.DMA,
            pltpu.SemaphoreType.DMA,
        ],
    )(in_emb, out_emb, bd2r, idx_in, idx_out, types, w0, w1, dn, rn, dc, rc)


def _logsig(x):
    return jnp.minimum(x, 0.0) - jnp.log(1.0 + jnp.exp(-jnp.abs(x)))


def _finish_body(zp_ref, w0_ref, w1_ref, sub0_ref, out_ref):
    b = pl.program_id(0)
    blk = zp_ref[...]
    ri = lax.broadcasted_iota(jnp.int32, (15 * 16, 16), 0)
    ci = lax.broadcasted_iota(jnp.int32, (15 * 16, 16), 1)
    sel = (ri // 16 == ci).astype(jnp.float32)
    z = jnp.dot(blk, sel, preferred_element_type=jnp.float32,
                precision=lax.Precision.HIGHEST)
    k = lax.broadcasted_iota(jnp.int32, (1, 16), 1)
    wlog = jnp.where(k == 0, 2.0, jnp.where(k < 11, 1.0, 0.0))
    wreg = jnp.where((k >= 11) & (k < 15), 1.0, 0.0)
    val = -jnp.sum(_logsig(z) * wlog) + jnp.sum(z * wreg)

    @pl.when(b == 0)
    def _():
        out_ref[...] = jnp.zeros_like(out_ref)

    out_ref[...] += val

    @pl.when(b == 7)
    def _():
        w0 = w0_ref[...]
        w1 = w1_ref[...]
        sub0 = sub0_ref[0, 0].astype(jnp.float32)
        wterm = (sub0 * jnp.sum(w0 * w0)
                 + (jnp.float32(BATCH) - sub0) * jnp.sum(w1 * w1))
        out_ref[...] = (out_ref[...] + wterm) / (2.0 * BATCH)


def _finish(zp2, w0, w1, sub0):
    return pl.pallas_call(
        _finish_body,
        grid=(8,),
        in_specs=[
            pl.BlockSpec((BATCH // 8, 15 * 16), lambda b: (b, 0)),
            pl.BlockSpec((1, EMBED), lambda b: (0, 0)),
            pl.BlockSpec((1, EMBED), lambda b: (0, 0)),
            pl.BlockSpec((1, 1), lambda b: (0, 0)),
        ],
        out_specs=pl.BlockSpec((1, 1), lambda b: (0, 0)),
        out_shape=jax.ShapeDtypeStruct((1, 1), jnp.float32),
    )(zp2, w0, w1, sub0)


def kernel(input_labels, out_labels, num_sampled, in_embed_w, out_embed_w,
           edge_w0, edge_w1):
    del num_sampled
    types = input_labels[:, 0]
    in_ids = input_labels[:, 1]
    out_t = out_labels[:, 1]

    dn, rn, dc, rc, sub0 = _prep(types.reshape(_RBLK, _CBLK))

    bd2r = jnp.asarray(_BD2R)
    zp = _sc_fused(in_embed_w, out_embed_w, bd2r, in_ids, out_t, types,
                   edge_w0, edge_w1, dn, rn, dc, rc)

    res = _finish(zp.reshape(BATCH, 15 * 16),
                  edge_w0.reshape(1, EMBED), edge_w1.reshape(1, EMBED), sub0)
    return res[0, 0]


# speedup vs baseline: 1.0511x; 1.0092x over previous
import numpy as np
import jax
import jax.numpy as jnp
from jax import lax
from jax.experimental import pallas as pl
from jax.experimental.pallas import tpu as pltpu
from jax.experimental.pallas import tpu_sc as plsc

EMBED = 64
BATCH = 4096
NSAMP = 5
RANGE_WIDTH = 50000
TOTAL_DRAWS = 2 * BATCH * NSAMP

NC = 2
NSUB = 16
NW = NC * NSUB
RPW = BATCH // NW
NPW = RPW * NSAMP

_BD = np.random.RandomState(0).randint(
    0, RANGE_WIDTH, size=(TOTAL_DRAWS,)).astype(np.int32)
_BD2R = np.concatenate([_BD, _BD + RANGE_WIDTH]).astype(np.int32).reshape(-1, 16)

_RBLK = 32
_CBLK = BATCH // _RBLK


def _prep_body(t_ref, dn_ref, rn_ref, dc_ref, rc_ref, sub0_ref):
    t = t_ref[...]
    m0 = (t == 0)
    m0f = m0.astype(jnp.float32)
    ri = lax.broadcasted_iota(jnp.int32, (_CBLK, _CBLK), 0)
    ci = lax.broadcasted_iota(jnp.int32, (_CBLK, _CBLK), 1)
    upper = (ri <= ci).astype(jnp.float32)
    crow = jnp.dot(m0f, upper, preferred_element_type=jnp.float32,
                   precision=lax.Precision.HIGHEST)
    s = crow[:, _CBLK - 1:_CBLK]
    ri2 = lax.broadcasted_iota(jnp.int32, (_RBLK, _RBLK), 0)
    ci2 = lax.broadcasted_iota(jnp.int32, (_RBLK, _RBLK), 1)
    lower = (ri2 > ci2).astype(jnp.float32)
    off = jnp.dot(lower, jnp.broadcast_to(s, (_RBLK, _CBLK)),
                  preferred_element_type=jnp.float32,
                  precision=lax.Precision.HIGHEST)[:, 0:1]
    cum = (off + crow).astype(jnp.int32)
    g0 = lax.broadcasted_iota(jnp.int32, (_RBLK, _CBLK), 0)
    g1 = lax.broadcasted_iota(jnp.int32, (_RBLK, _CBLK), 1)
    gidx = g0 * _CBLK + g1
    rank = jnp.where(m0, cum - 1, gidx - cum)
    sub0 = cum[_RBLK - 1:_RBLK, _CBLK - 1:_CBLK]
    n0s = sub0 * NSAMP
    n1s = (BATCH - sub0) * NSAMP
    noise_start = jnp.where(m0, 0, 2 * n0s)
    cp_start = jnp.where(m0, n0s, 2 * n0s + n1s)
    posn = (noise_start + rank * NSAMP).astype(jnp.float32)
    posc = (cp_start + rank * NSAMP).astype(jnp.float32)
    tf = t.astype(jnp.float32)

    re = lax.broadcasted_iota(jnp.int32, (_RBLK * NSAMP, _RBLK), 0)
    ce = lax.broadcasted_iota(jnp.int32, (_RBLK * NSAMP, _RBLK), 1)
    exp_mat = (re // NSAMP == ce).astype(jnp.float32)
    svec = (lax.broadcasted_iota(jnp.int32, (_RBLK * NSAMP, _CBLK), 0)
            % NSAMP).astype(jnp.float32)
    posn3 = jnp.dot(exp_mat, posn, preferred_element_type=jnp.float32,
                    precision=lax.Precision.HIGHEST) + svec
    posc3 = jnp.dot(exp_mat, posc, preferred_element_type=jnp.float32,
                    precision=lax.Precision.HIGHEST) + svec
    t3 = jnp.dot(exp_mat, tf, preferred_element_type=jnp.float32,
                    precision=lax.Precision.HIGHEST)
    pn = (posn3 + TOTAL_DRAWS * t3).astype(jnp.int32)
    pc = (posc3 + TOTAL_DRAWS * (1.0 - t3)).astype(jnp.int32)
    dn_ref[...] = pn // 16
    rn_ref[...] = pn % 16
    dc_ref[...] = pc // 16
    rc_ref[...] = pc % 16
    sub0_ref[...] = sub0


def _prep(types2d):
    i32 = jnp.int32
    return pl.pallas_call(
        _prep_body,
        out_shape=(
            jax.ShapeDtypeStruct((_RBLK * NSAMP, _CBLK), i32),
            jax.ShapeDtypeStruct((_RBLK * NSAMP, _CBLK), i32),
            jax.ShapeDtypeStruct((_RBLK * NSAMP, _CBLK), i32),
            jax.ShapeDtypeStruct((_RBLK * NSAMP, _CBLK), i32),
            jax.ShapeDtypeStruct((1, 1), i32),
        ),
    )(types2d)


def _sc_body(in_emb, out_emb, bd2_hbm, idx_in_hbm, idx_out_hbm, types_hbm,
             w0_hbm, w1_hbm, dn_hbm, rn_hbm, dc_hbm, rc_hbm,
             zp_hbm,
             dn_v, rn_v, dc_v, rc_v, idxn_v, idxc_v, idxmi_v, idxmo_v,
             types_v, w0_v, w1_v, bdr_n, bdr_c, rows_mi, rows_mo,
             rows_n, rows_c, zp_v,
             sem_s, sem_b, sem_r, semn0, semn1, semc0, semc1):
    c = lax.axis_index("c")
    s = lax.axis_index("s")
    wid = s * NC + c
    base = wid * RPW

    ph0 = [
        pltpu.async_copy(dn_hbm.at[pl.ds(wid * NSAMP, NSAMP)], dn_v, sem_s),
        pltpu.async_copy(rn_hbm.at[pl.ds(wid * NSAMP, NSAMP)], rn_v, sem_s),
        pltpu.async_copy(dc_hbm.at[pl.ds(wid * NSAMP, NSAMP)], dc_v, sem_s),
        pltpu.async_copy(rc_hbm.at[pl.ds(wid * NSAMP, NSAMP)], rc_v, sem_s),
        pltpu.async_copy(idx_in_hbm.at[pl.ds(base, RPW)], idxmi_v, sem_s),
        pltpu.async_copy(idx_out_hbm.at[pl.ds(base, RPW)], idxmo_v, sem_s),
        pltpu.async_copy(types_hbm.at[pl.ds(base, RPW)], types_v, sem_s),
        pltpu.async_copy(w0_hbm, w0_v, sem_s),
        pltpu.async_copy(w1_hbm, w1_v, sem_s),
    ]
    for cp in ph0:
        cp.wait()
    main_cps = [
        pltpu.async_copy(in_emb.at[idxmi_v], rows_mi, sem_r),
        pltpu.async_copy(out_emb.at[idxmo_v], rows_mo, sem_r),
    ]
    bd_cps = [pltpu.async_copy(bd2_hbm.at[dn_v.at[sl]], bdr_n.at[sl], sem_b)
              for sl in range(NSAMP)]
    bd_cps += [pltpu.async_copy(bd2_hbm.at[dc_v.at[sl]], bdr_c.at[sl], sem_b)
               for sl in range(NSAMP)]
    for cp in bd_cps:
        cp.wait()
    lane = lax.broadcasted_iota(jnp.int32, (16,), 0)
    for sl in range(NSAMP):
        for ch in range(RPW // 16):
            sl16 = jnp.zeros((16,), jnp.int32) + sl
            idxn_v[sl, pl.ds(ch * 16, 16)] = plsc.load_gather(
                bdr_n, [sl16, lane + ch * 16, rn_v[sl, pl.ds(ch * 16, 16)]])
            idxc_v[sl, pl.ds(ch * 16, 16)] = plsc.load_gather(
                bdr_c, [sl16, lane + ch * 16, rc_v[sl, pl.ds(ch * 16, 16)]])

    semn = [semn0, semn1]
    semc = [semc0, semc1]

    def fire_n(sl):
        return pltpu.async_copy(in_emb.at[idxn_v.at[sl]],
                                rows_n.at[sl % 2], semn[sl % 2])

    def fire_c(sl):
        return pltpu.async_copy(out_emb.at[idxc_v.at[sl]],
                                rows_c.at[sl % 2], semc[sl % 2])

    ncps = [fire_n(0), fire_n(1)]
    ccps = [fire_c(0), fire_c(1)]
    for cp in main_cps:
        cp.wait()

    w0c = [w0_v[pl.ds(ci * 16, 16)] for ci in range(EMBED // 16)]
    w1c = [w1_v[pl.ds(ci * 16, 16)] for ci in range(EMBED // 16)]
    zero = jnp.zeros((16,), jnp.float32)
    nchunk = EMBED // 16

    def wsel_chunks(r):
        tm = plsc.load_gather(types_v, [jnp.zeros((16,), jnp.int32) + r])
        return [jnp.where(tm != 0, w1c[ci], w0c[ci]) for ci in range(nchunk)]

    def pass_a(r, carry):
        ws = wsel_chunks(r)
        zt = zero
        ninp = zero
        noutp = zero
        for ci in range(nchunk):
            ic = rows_mi[r, pl.ds(ci * 16, 16)]
            oc = rows_mo[r, pl.ds(ci * 16, 16)]
            zt = zt + ic * (oc * ws[ci])
            ninp = ninp + ic * ic
            noutp = noutp + oc * oc
        zp_v[r, 0, :] = zt
        zp_v[r, 11, :] = ninp
        zp_v[r, 12, :] = noutp
        zp_v[r, 13, :] = zero
        zp_v[r, 14, :] = zero
        return carry

    lax.fori_loop(0, RPW, pass_a, 0)

    def make_slab_n(sl, buf):
        def body(r, carry):
            ws = wsel_chunks(r)
            zu = zero
            nn = zero
            for ci in range(nchunk):
                nc = rows_n[buf, r, pl.ds(ci * 16, 16)]
                oc = rows_mo[r, pl.ds(ci * 16, 16)]
                zu = zu + nc * (oc * ws[ci])
                nn = nn + nc * nc
            zp_v[r, 1 + sl, :] = zu
            zp_v[r, 13, :] = zp_v[r, 13, :] + nn
            return carry
        return body

    def make_slab_c(sl, buf):
        def body(r, carry):
            ws = wsel_chunks(r)
            zv = zero
            ncp = zero
            for ci in range(nchunk):
                cc = rows_c[buf, r, pl.ds(ci * 16, 16)]
                ic = rows_mi[r, pl.ds(ci * 16, 16)]
                zv = zv + cc * (ic * ws[ci])
                ncp = ncp + cc * cc
            zp_v[r, 6 + sl, :] = zv
            zp_v[r, 14, :] = zp_v[r, 14, :] + ncp
            return carry
        return body

    for sl in range(NSAMP):
        ncps[sl].wait()
        lax.fori_loop(0, RPW, make_slab_n(sl, sl % 2), 0)
        if sl + 2 < NSAMP:
            ncps.append(fire_n(sl + 2))
        ccps[sl].wait()
        lax.fori_loop(0, RPW, make_slab_c(sl, sl % 2), 0)
        if sl + 2 < NSAMP:
            ccps.append(fire_c(sl + 2))

    pltpu.sync_copy(zp_v, zp_hbm.at[wid])


def _sc_fused(in_emb, out_emb, bd2r, idx_in, idx_out, types, w0, w1,
              dn, rn, dc, rc):
    f32 = jnp.float32
    i32 = jnp.int32
    return pl.kernel(
        _sc_body,
        out_type=jax.ShapeDtypeStruct((NW, RPW, 15, 16), f32),
        mesh=plsc.VectorSubcoreMesh(core_axis_name="c", subcore_axis_name="s"),
        compiler_params=pltpu.CompilerParams(needs_layout_passes=False,
                                             use_tc_tiling_on_sc=False),
        scratch_types=[
            pltpu.VMEM((NSAMP, RPW), i32),
            pltpu.VMEM((NSAMP, RPW), i32),
            pltpu.VMEM((NSAMP, RPW), i32),
            pltpu.VMEM((NSAMP, RPW), i32),
            pltpu.VMEM((NSAMP, RPW), i32),
            pltpu.VMEM((NSAMP, RPW), i32),
            pltpu.VMEM((RPW,), i32),
            pltpu.VMEM((RPW,), i32),
            pltpu.VMEM((RPW,), i32),
            pltpu.VMEM((EMBED,), f32),
            pltpu.VMEM((EMBED,), f32),
            pltpu.VMEM((NSAMP, RPW, 16), i32),
            pltpu.VMEM((NSAMP, RPW, 16), i32),
            pltpu.VMEM((RPW, EMBED), f32),
            pltpu.VMEM((RPW, EMBED), f32),
            pltpu.VMEM((2, RPW, EMBED), f32),
            pltpu.VMEM((2, RPW, EMBED), f32),
            pltpu.VMEM((RPW, 15, 16), f32),
            pltpu.SemaphoreType.DMA,
            pltpu.SemaphoreType.DMA,
            pltpu.SemaphoreType.DMA,
            pltpu.SemaphoreType.DMA,
            pltpu.SemaphoreType.DMA,
            pltpu.SemaphoreType.DMA,
            pltpu.SemaphoreType.DMA,
        ],
    )(in_emb, out_emb, bd2r, idx_in, idx_out, types, w0, w1, dn, rn, dc, rc)


def _logsig(x):
    return jnp.minimum(x, 0.0) - jnp.log(1.0 + jnp.exp(-jnp.abs(x)))


def _finish_body(zp_ref, w0_ref, w1_ref, sub0_ref, out_ref):
    b = pl.program_id(0)
    blk = zp_ref[...]
    ri = lax.broadcasted_iota(jnp.int32, (15 * 16, 16), 0)
    ci = lax.broadcasted_iota(jnp.int32, (15 * 16, 16), 1)
    sel = (ri // 16 == ci).astype(jnp.float32)
    z = jnp.dot(blk, sel, preferred_element_type=jnp.float32,
                precision=lax.Precision.HIGHEST)
    k = lax.broadcasted_iota(jnp.int32, (1, 16), 1)
    wlog = jnp.where(k == 0, 2.0, jnp.where(k < 11, 1.0, 0.0))
    wreg = jnp.where((k >= 11) & (k < 15), 1.0, 0.0)
    val = -jnp.sum(_logsig(z) * wlog) + jnp.sum(z * wreg)

    @pl.when(b == 0)
    def _():
        out_ref[...] = jnp.zeros_like(out_ref)

    out_ref[...] += val

    @pl.when(b == 3)
    def _():
        w0 = w0_ref[...]
        w1 = w1_ref[...]
        sub0 = sub0_ref[0, 0].astype(jnp.float32)
        wterm = (sub0 * jnp.sum(w0 * w0)
                 + (jnp.float32(BATCH) - sub0) * jnp.sum(w1 * w1))
        out_ref[...] = (out_ref[...] + wterm) / (2.0 * BATCH)


def _finish(zp2, w0, w1, sub0):
    return pl.pallas_call(
        _finish_body,
        grid=(4,),
        in_specs=[
            pl.BlockSpec((BATCH // 4, 15 * 16), lambda b: (b, 0)),
            pl.BlockSpec((1, EMBED), lambda b: (0, 0)),
            pl.BlockSpec((1, EMBED), lambda b: (0, 0)),
            pl.BlockSpec((1, 1), lambda b: (0, 0)),
        ],
        out_specs=pl.BlockSpec((1, 1), lambda b: (0, 0)),
        out_shape=jax.ShapeDtypeStruct((1, 1), jnp.float32),
    )(zp2, w0, w1, sub0)


def kernel(input_labels, out_labels, num_sampled, in_embed_w, out_embed_w,
           edge_w0, edge_w1):
    del num_sampled
    types = input_labels[:, 0]
    in_ids = input_labels[:, 1]
    out_t = out_labels[:, 1]

    dn, rn, dc, rc, sub0 = _prep(types.reshape(_RBLK, _CBLK))

    bd2r = jnp.asarray(_BD2R)
    zp = _sc_fused(in_embed_w, out_embed_w, bd2r, in_ids, out_t, types,
                   edge_w0, edge_w1, dn, rn, dc, rc)

    res = _finish(zp.reshape(BATCH, 15 * 16),
                  edge_w0.reshape(1, EMBED), edge_w1.reshape(1, EMBED), sub0)
    return res[0, 0]
